# baseline (speedup 1.0000x reference)
import functools

import jax
import jax.numpy as jnp
import numpy as np
from jax import lax
from jax.experimental import pallas as pl
from jax.experimental.pallas import tpu as pltpu
from jax.experimental.pallas import tpu_sc as plsc

N = 2048
NW = 32
BLK = 32
NBLK = N // BLK
L = 16


def _tf2x32(k1v, k2v, c1, c2):
    R0 = [13, 15, 26, 6]
    R1 = [17, 29, 16, 24]
    ks0 = np.asarray(k1v, np.uint32)
    ks1 = np.asarray(k2v, np.uint32)
    ks = [ks0, ks1, (ks0 ^ ks1 ^ np.uint32(0x1BD11BDA)).astype(np.uint32)]
    x0 = (c1.astype(np.uint32) + ks[0]).astype(np.uint32)
    x1 = (c2.astype(np.uint32) + ks[1]).astype(np.uint32)

    def rol(x, d):
        return ((x << np.uint32(d)) | (x >> np.uint32(32 - d))).astype(np.uint32)

    with np.errstate(over="ignore"):
        for i in range(5):
            for r in (R0 if i % 2 == 0 else R1):
                x0 = (x0 + x1).astype(np.uint32)
                x1 = rol(x1, r)
                x1 = (x1 ^ x0).astype(np.uint32)
            x0 = (x0 + ks[(i + 1) % 3]).astype(np.uint32)
            x1 = (x1 + ks[(i + 2) % 3] + np.uint32(i + 1)).astype(np.uint32)
    return x0, x1


def _uniform01(bits):
    f = ((bits >> np.uint32(9)) | np.uint32(0x3F800000)).view(np.float32)
    return np.maximum(np.float32(0.0), (f - np.float32(1.0)).astype(np.float32))


@functools.lru_cache(maxsize=1)
def _constants():
    idx = np.arange(N * N, dtype=np.uint64)
    b1, b2 = _tf2x32(0, 1, (idx >> np.uint64(32)).astype(np.uint32),
                     idx.astype(np.uint32))
    gate = (_uniform01((b1 ^ b2).astype(np.uint32)) <
            np.float32(0.5)).reshape(N, N)
    ii = np.arange(N)
    g = gate & (ii[:, None] > ii[None, :])
    gw = np.zeros((NBLK, N), np.uint32)
    for r in range(BLK):
        gw |= g[r::BLK, :].astype(np.uint32) << r
    kidx = np.arange(N, dtype=np.uint64)
    kb1, kb2 = _tf2x32(0, 2, (kidx >> np.uint64(32)).astype(np.uint32),
                       kidx.astype(np.uint32))
    zb = np.zeros(N, np.uint32)
    ub1, ub2 = _tf2x32(kb1, kb2, zb, zb)
    u = _uniform01((ub1 ^ ub2).astype(np.uint32))
    return gw.view(np.int32).reshape(-1), u


_GW_CONST, _U_CONST = _constants()


def _sc_body(colpx_h, colpy_h, colrad_h, u_h, gw_h,
             outx_h, outy_h,
             colpx, colpy, colrad, uown, gwv, maskbuf, jbuf,
             outv_x, outv_y, sem):
    cid = lax.axis_index("c")
    sid = lax.axis_index("s")
    wid = sid * 2 + cid

    pltpu.sync_copy(colpx_h, colpx)
    pltpu.sync_copy(colpy_h, colpy)
    pltpu.sync_copy(colrad_h, colrad)

    iota = lax.iota(jnp.int32, L)

    def do_block(half, blk):
        base = blk * BLK
        pltpu.sync_copy(gw_h.at[pl.ds(blk * N, N)],
                        gwv.at[pl.ds(half * N, N)])
        pltpu.sync_copy(u_h.at[pl.ds(base, BLK)],
                        uown.at[pl.ds(half * BLK, BLK)])

        def row_body(r, _):
            i = base + r
            nch = (i + L - 1) // L
            ngrp = (nch + 3) // 4
            isplat = jnp.full((L,), 0, jnp.int32) + i
            bpx = plsc.load_gather(colpx, [isplat])
            bpy = plsc.load_gather(colpy, [isplat])
            brad = plsc.load_gather(colrad, [isplat])
            gbit = jnp.full((L,), 0, jnp.int32) + lax.shift_left(
                jnp.int32(1), r)

            def pa(g, acc):
                for k in range(4):
                    off = g * (4 * L) + k * L
                    px16 = colpx[pl.ds(off, L)]
                    py16 = colpy[pl.ds(off, L)]
                    rad16 = colrad[pl.ds(off, L)]
                    dx = px16 - bpx
                    dy = py16 - bpy
                    d2e = dx * dx + dy * dy + 1e-12
                    rs = rad16 + brad
                    geom = d2e < rs * rs
                    gvec = gwv[pl.ds(half * N + off, L)]
                    gb = (gvec & gbit) != 0
                    m = jnp.where(geom & gb, 1, 0).astype(jnp.int32)
                    maskbuf[pl.ds(off, L)] = m
                    acc = acc + m
                return acc

            cntv = lax.fori_loop(0, ngrp, pa, jnp.zeros((L,), jnp.int32))
            cnt_s = jnp.sum(cntv)

            def pass_b():
                cnt_f = cnt_s.astype(jnp.float32)
                qv = jnp.full((L,), 1.0, jnp.float32) / cnt_f

                tcv = lax.fori_loop(1, cnt_s + 1, lambda k, t: t + qv,
                                    jnp.zeros((L,), jnp.float32))
                lsplat = jnp.full((L,), 0, jnp.int32) + (half * BLK + r)
                uvec = plsc.load_gather(uown, [lsplat])
                rv = tcv * (1.0 - uvec)

                def l2(k, carry):
                    t, m = carry
                    t2 = t + qv
                    m2 = m + jnp.where(t2 < rv, 1, 0).astype(jnp.int32)
                    return t2, m2

                _, mv = lax.fori_loop(1, cnt_s + 1, l2,
                                      (jnp.zeros((L,), jnp.float32),
                                       jnp.zeros((L,), jnp.int32)))
                target_s = jnp.max(mv) + 1

                def gcond(st):
                    g, bacc, _ = st
                    return (g < ngrp) & (bacc < target_s)

                def gbody(st):
                    g, bacc, _ = st
                    off = g * (4 * L)
                    s = (maskbuf[pl.ds(off, L)] +
                         maskbuf[pl.ds(off + L, L)] +
                         maskbuf[pl.ds(off + 2 * L, L)] +
                         maskbuf[pl.ds(off + 3 * L, L)])
                    return g + 1, bacc + jnp.sum(s), bacc

                gend, _, bprev = lax.while_loop(
                    gcond, gbody, (jnp.int32(0), jnp.int32(0), jnp.int32(0)))

                def cond(st):
                    c, bacc, _ = st
                    return (c < 4 * ngrp) & (bacc < target_s)

                def pb(st):
                    c, bacc, jacc = st
                    off = c * L
                    mvec = maskbuf[pl.ds(off, L)]
                    pcs = plsc.cumsum(mvec)
                    ind = ((pcs + bacc) == target_s) & (mvec > 0)
                    jhit = jnp.sum(jnp.where(ind, iota + off, 0))
                    bsum = jnp.sum(mvec)
                    return c + 1, bacc + bsum, jacc + jhit

                _, _, j = lax.while_loop(cond, pb,
                                         ((gend - 1) * 4, bprev, jnp.int32(0)))
                return j

            j_s = lax.cond(cnt_s > 0, pass_b, lambda: jnp.int32(-1))

            lsplat2 = jnp.full((L,), 0, jnp.int32) + (half * BLK + r)
            plsc.store_scatter(jbuf, [lsplat2],
                               jnp.full((L,), 0, jnp.int32) + j_s,
                               mask=iota == 0)
            return 0

        lax.fori_loop(0, BLK, row_body, 0)

        for gch in range(BLK // L):
            lo = half * BLK + gch * L
            jv = jbuf[pl.ds(lo, L)]
            have = jv >= 0
            jc = jnp.maximum(jv, 0)
            pxj = plsc.load_gather(colpx, [jc])
            pyj = plsc.load_gather(colpy, [jc])
            rj = plsc.load_gather(colrad, [jc])
            pxi = colpx[pl.ds(base + gch * L, L)]
            pyi = colpy[pl.ds(base + gch * L, L)]
            ri = colrad[pl.ds(base + gch * L, L)]
            dx = pxj - pxi
            dy = pyj - pyi
            d2e = dx * dx + dy * dy + 1e-12
            bits = plsc.bitcast(d2e, jnp.int32)
            s = plsc.bitcast(
                lax.shift_right_logical(bits, 1) + jnp.int32(0x1FBD1DF5),
                jnp.float32)
            for _ in range(4):
                s = 0.5 * (s + d2e / s)
            depth = (ri + rj) - s
            outv_x[pl.ds(lo, L)] = jnp.where(have, (dx / s) * depth, 0.0)
            outv_y[pl.ds(lo, L)] = jnp.where(have, (dy / s) * depth, 0.0)

        pltpu.sync_copy(outv_x.at[pl.ds(half * BLK, BLK)],
                        outx_h.at[pl.ds(base, BLK)])
        pltpu.sync_copy(outv_y.at[pl.ds(half * BLK, BLK)],
                        outy_h.at[pl.ds(base, BLK)])

    do_block(0, wid)
    do_block(1, NBLK - 1 - wid)
    del sem


def kernel(positions, radii):
    gw = jnp.asarray(_GW_CONST)
    u = jnp.asarray(_U_CONST)
    colpx = positions[:, 0]
    colpy = positions[:, 1]

    mesh = plsc.VectorSubcoreMesh(core_axis_name="c", subcore_axis_name="s")
    f = pl.kernel(
        _sc_body,
        out_type=[
            jax.ShapeDtypeStruct((N,), jnp.float32),
            jax.ShapeDtypeStruct((N,), jnp.float32),
        ],
        mesh=mesh,
        compiler_params=pltpu.CompilerParams(needs_layout_passes=False),
        scratch_types=[
            pltpu.VMEM((N,), jnp.float32),
            pltpu.VMEM((N,), jnp.float32),
            pltpu.VMEM((N,), jnp.float32),
            pltpu.VMEM((2 * BLK,), jnp.float32),
            pltpu.VMEM((2 * N,), jnp.int32),
            pltpu.VMEM((N + 4 * L,), jnp.int32),
            pltpu.VMEM((2 * BLK,), jnp.int32),
            pltpu.VMEM((2 * BLK,), jnp.float32),
            pltpu.VMEM((2 * BLK,), jnp.float32),
            pltpu.SemaphoreType.DMA,
        ],
    )
    outx, outy = f(colpx, colpy, radii, u, gw)
    return jnp.stack([outx, outy], axis=1)


# speedup vs baseline: 1.0277x; 1.0277x over previous
import functools

import jax
import jax.numpy as jnp
import numpy as np
from jax import lax
from jax.experimental import pallas as pl
from jax.experimental.pallas import tpu as pltpu
from jax.experimental.pallas import tpu_sc as plsc

N = 2048
NW = 32
BLK = 32
NBLK = N // BLK
L = 16


def _tf2x32(k1v, k2v, c1, c2):
    R0 = [13, 15, 26, 6]
    R1 = [17, 29, 16, 24]
    ks0 = np.asarray(k1v, np.uint32)
    ks1 = np.asarray(k2v, np.uint32)
    ks = [ks0, ks1, (ks0 ^ ks1 ^ np.uint32(0x1BD11BDA)).astype(np.uint32)]
    x0 = (c1.astype(np.uint32) + ks[0]).astype(np.uint32)
    x1 = (c2.astype(np.uint32) + ks[1]).astype(np.uint32)

    def rol(x, d):
        return ((x << np.uint32(d)) | (x >> np.uint32(32 - d))).astype(np.uint32)

    with np.errstate(over="ignore"):
        for i in range(5):
            for r in (R0 if i % 2 == 0 else R1):
                x0 = (x0 + x1).astype(np.uint32)
                x1 = rol(x1, r)
                x1 = (x1 ^ x0).astype(np.uint32)
            x0 = (x0 + ks[(i + 1) % 3]).astype(np.uint32)
            x1 = (x1 + ks[(i + 2) % 3] + np.uint32(i + 1)).astype(np.uint32)
    return x0, x1


def _uniform01(bits):
    f = ((bits >> np.uint32(9)) | np.uint32(0x3F800000)).view(np.float32)
    return np.maximum(np.float32(0.0), (f - np.float32(1.0)).astype(np.float32))


@functools.lru_cache(maxsize=1)
def _constants():
    idx = np.arange(N * N, dtype=np.uint64)
    b1, b2 = _tf2x32(0, 1, (idx >> np.uint64(32)).astype(np.uint32),
                     idx.astype(np.uint32))
    gate = (_uniform01((b1 ^ b2).astype(np.uint32)) <
            np.float32(0.5)).reshape(N, N)
    ii = np.arange(N)
    g = gate & (ii[:, None] > ii[None, :])
    gw = np.zeros((NBLK, N), np.uint32)
    for r in range(BLK):
        gw |= g[r::BLK, :].astype(np.uint32) << r
    kidx = np.arange(N, dtype=np.uint64)
    kb1, kb2 = _tf2x32(0, 2, (kidx >> np.uint64(32)).astype(np.uint32),
                       kidx.astype(np.uint32))
    zb = np.zeros(N, np.uint32)
    ub1, ub2 = _tf2x32(kb1, kb2, zb, zb)
    u = _uniform01((ub1 ^ ub2).astype(np.uint32))
    return gw.view(np.int32).reshape(-1), u


_GW_CONST, _U_CONST = _constants()


def _sc_body(colpx_h, colpy_h, colrad_h, u_h, gw_h,
             outx_h, outy_h,
             colpx, colpy, colrad, uown, gwv, maskbuf, jbuf,
             outv_x, outv_y, sem):
    cid = lax.axis_index("c")
    sid = lax.axis_index("s")
    wid = sid * 2 + cid
    blk_a = wid
    blk_b = NBLK - 1 - wid

    copies = [
        pltpu.async_copy(colpx_h, colpx, sem),
        pltpu.async_copy(colpy_h, colpy, sem),
        pltpu.async_copy(colrad_h, colrad, sem),
        pltpu.async_copy(gw_h.at[pl.ds(blk_a * N, N)],
                         gwv.at[pl.ds(0, N)], sem),
        pltpu.async_copy(gw_h.at[pl.ds(blk_b * N, N)],
                         gwv.at[pl.ds(N, N)], sem),
        pltpu.async_copy(u_h.at[pl.ds(blk_a * BLK, BLK)],
                         uown.at[pl.ds(0, BLK)], sem),
        pltpu.async_copy(u_h.at[pl.ds(blk_b * BLK, BLK)],
                         uown.at[pl.ds(BLK, BLK)], sem),
    ]
    for c in copies:
        c.wait()

    iota = lax.iota(jnp.int32, L)

    def do_block(half, blk):
        base = blk * BLK

        def row_body(r, _):
            i = base + r
            nch = (i + L - 1) // L
            ngrp = (nch + 3) // 4
            isplat = jnp.full((L,), 0, jnp.int32) + i
            bpx = plsc.load_gather(colpx, [isplat])
            bpy = plsc.load_gather(colpy, [isplat])
            brad = plsc.load_gather(colrad, [isplat])
            gbit = jnp.full((L,), 0, jnp.int32) + lax.shift_left(
                jnp.int32(1), r)

            def pa(g, acc):
                for k in range(4):
                    off = g * (4 * L) + k * L
                    px16 = colpx[pl.ds(off, L)]
                    py16 = colpy[pl.ds(off, L)]
                    rad16 = colrad[pl.ds(off, L)]
                    dx = px16 - bpx
                    dy = py16 - bpy
                    d2e = dx * dx + dy * dy + 1e-12
                    rs = rad16 + brad
                    geom = d2e < rs * rs
                    gvec = gwv[pl.ds(half * N + off, L)]
                    gb = (gvec & gbit) != 0
                    m = jnp.where(geom & gb, 1, 0).astype(jnp.int32)
                    maskbuf[pl.ds(off, L)] = m
                    acc = acc + m
                return acc

            cntv = lax.fori_loop(0, ngrp, pa, jnp.zeros((L,), jnp.int32))
            cnt_s = jnp.sum(cntv)

            def pass_b():
                cnt_f = cnt_s.astype(jnp.float32)
                qv = jnp.full((L,), 1.0, jnp.float32) / cnt_f

                tcv = lax.fori_loop(1, cnt_s + 1, lambda k, t: t + qv,
                                    jnp.zeros((L,), jnp.float32))
                lsplat = jnp.full((L,), 0, jnp.int32) + (half * BLK + r)
                uvec = plsc.load_gather(uown, [lsplat])
                rv = tcv * (1.0 - uvec)

                def l2(k, carry):
                    t, m = carry
                    t2 = t + qv
                    m2 = m + jnp.where(t2 < rv, 1, 0).astype(jnp.int32)
                    return t2, m2

                _, mv = lax.fori_loop(1, cnt_s + 1, l2,
                                      (jnp.zeros((L,), jnp.float32),
                                       jnp.zeros((L,), jnp.int32)))
                target_s = jnp.max(mv) + 1

                def gcond(st):
                    g, bacc, _ = st
                    return (g < ngrp) & (bacc < target_s)

                def gbody(st):
                    g, bacc, _ = st
                    off = g * (4 * L)
                    s = (maskbuf[pl.ds(off, L)] +
                         maskbuf[pl.ds(off + L, L)] +
                         maskbuf[pl.ds(off + 2 * L, L)] +
                         maskbuf[pl.ds(off + 3 * L, L)])
                    return g + 1, bacc + jnp.sum(s), bacc

                gend, _, bprev = lax.while_loop(
                    gcond, gbody, (jnp.int32(0), jnp.int32(0), jnp.int32(0)))

                def cond(st):
                    c, bacc, _ = st
                    return (c < 4 * ngrp) & (bacc < target_s)

                def pb(st):
                    c, bacc, jacc = st
                    off = c * L
                    mvec = maskbuf[pl.ds(off, L)]
                    pcs = plsc.cumsum(mvec)
                    ind = ((pcs + bacc) == target_s) & (mvec > 0)
                    jhit = jnp.sum(jnp.where(ind, iota + off, 0))
                    bsum = jnp.sum(mvec)
                    return c + 1, bacc + bsum, jacc + jhit

                _, _, j = lax.while_loop(cond, pb,
                                         ((gend - 1) * 4, bprev, jnp.int32(0)))
                return j

            j_s = lax.cond(cnt_s > 0, pass_b, lambda: jnp.int32(-1))

            lsplat2 = jnp.full((L,), 0, jnp.int32) + (half * BLK + r)
            plsc.store_scatter(jbuf, [lsplat2],
                               jnp.full((L,), 0, jnp.int32) + j_s,
                               mask=iota == 0)
            return 0

        lax.fori_loop(0, BLK, row_body, 0)

        for gch in range(BLK // L):
            lo = half * BLK + gch * L
            jv = jbuf[pl.ds(lo, L)]
            have = jv >= 0
            jc = jnp.maximum(jv, 0)
            pxj = plsc.load_gather(colpx, [jc])
            pyj = plsc.load_gather(colpy, [jc])
            rj = plsc.load_gather(colrad, [jc])
            pxi = colpx[pl.ds(base + gch * L, L)]
            pyi = colpy[pl.ds(base + gch * L, L)]
            ri = colrad[pl.ds(base + gch * L, L)]
            dx = pxj - pxi
            dy = pyj - pyi
            d2e = dx * dx + dy * dy + 1e-12
            bits = plsc.bitcast(d2e, jnp.int32)
            s = plsc.bitcast(
                lax.shift_right_logical(bits, 1) + jnp.int32(0x1FBD1DF5),
                jnp.float32)
            for _ in range(4):
                s = 0.5 * (s + d2e / s)
            depth = (ri + rj) - s
            outv_x[pl.ds(lo, L)] = jnp.where(have, (dx / s) * depth, 0.0)
            outv_y[pl.ds(lo, L)] = jnp.where(have, (dy / s) * depth, 0.0)

        pltpu.sync_copy(outv_x.at[pl.ds(half * BLK, BLK)],
                        outx_h.at[pl.ds(base, BLK)])
        pltpu.sync_copy(outv_y.at[pl.ds(half * BLK, BLK)],
                        outy_h.at[pl.ds(base, BLK)])

    do_block(0, blk_a)
    do_block(1, blk_b)


def kernel(positions, radii):
    gw = jnp.asarray(_GW_CONST)
    u = jnp.asarray(_U_CONST)
    colpx = positions[:, 0]
    colpy = positions[:, 1]

    mesh = plsc.VectorSubcoreMesh(core_axis_name="c", subcore_axis_name="s")
    f = pl.kernel(
        _sc_body,
        out_type=[
            jax.ShapeDtypeStruct((N,), jnp.float32),
            jax.ShapeDtypeStruct((N,), jnp.float32),
        ],
        mesh=mesh,
        compiler_params=pltpu.CompilerParams(needs_layout_passes=False),
        scratch_types=[
            pltpu.VMEM((N,), jnp.float32),
            pltpu.VMEM((N,), jnp.float32),
            pltpu.VMEM((N,), jnp.float32),
            pltpu.VMEM((2 * BLK,), jnp.float32),
            pltpu.VMEM((2 * N,), jnp.int32),
            pltpu.VMEM((N + 4 * L,), jnp.int32),
            pltpu.VMEM((2 * BLK,), jnp.int32),
            pltpu.VMEM((2 * BLK,), jnp.float32),
            pltpu.VMEM((2 * BLK,), jnp.float32),
            pltpu.SemaphoreType.DMA,
        ],
    )
    outx, outy = f(colpx, colpy, radii, u, gw)
    return jnp.stack([outx, outy], axis=1)
